# baseline (speedup 1.0000x reference)
import jax
import jax.numpy as jnp
from jax import lax
from jax.experimental import pallas as pl
from jax.experimental.pallas import tpu as pltpu
from jax.experimental.pallas import tpu_sc as plsc

N_NODES_K = 10000
N_OUT_K = 6000
GDIM = 128
E_K = 320000
NC_K, NS_K = 2, 16
NW_K = NC_K * NS_K
CHUNK_K = 32
CHUNK_SHIFT_K = 5
NBUF_K = 7
GDEPTH_K = 6
EDGES_PER_TILE_K = E_K // NW_K
ROW_K = 80
N_ROWS_K = EDGES_PER_TILE_K // ROW_K
SP_ROWS_K = 6016
STRIPE_K = SP_ROWS_K // NS_K
TRASH_K = 6016
DEG_W_K = 6032
N_PAD_K = 6016
OUT_STRIPE_K = N_PAD_K // NS_K


_PROJ_BLOCK = 1000


def _proj_body(xc_ref, xp_ref, wc_ref, bc_ref, wp_ref, bp_ref, o_ref):
    i = pl.program_id(0)

    @pl.when(i < 6)
    def _():
        o_ref[...] = jnp.dot(xc_ref[...], wc_ref[...],
                             preferred_element_type=jnp.float32) + bc_ref[...]

    @pl.when(i >= 6)
    def _():
        o_ref[...] = jnp.dot(xp_ref[...], wp_ref[...],
                             preferred_element_type=jnp.float32) + bp_ref[...]


def _proj(x_customer, x_product, lin_c_w, lin_c_b, lin_p_w, lin_p_b):
    full = pl.BlockSpec((GDIM, GDIM), lambda i: (0, 0))
    bias = pl.BlockSpec((1, GDIM), lambda i: (0, 0))
    return pl.pallas_call(
        _proj_body,
        grid=(N_NODES_K // _PROJ_BLOCK,),
        in_specs=[
            pl.BlockSpec((_PROJ_BLOCK, GDIM), lambda i: (jnp.minimum(i, 5), 0)),
            pl.BlockSpec((_PROJ_BLOCK, GDIM), lambda i: (jnp.maximum(i - 6, 0), 0)),
            full, bias, full, bias,
        ],
        out_specs=pl.BlockSpec((_PROJ_BLOCK, GDIM), lambda i: (i, 0)),
        out_shape=jax.ShapeDtypeStruct((N_NODES_K, GDIM), jnp.float32),
    )(x_customer, x_product, lin_c_w, lin_c_b.reshape(1, GDIM),
      lin_p_w, lin_p_b.reshape(1, GDIM))


def _sc_body(x_hbm, src_hbm, dst_hbm, agg_out, deg_out,
             src_b, dst_b, ids_b, *rest):
    csd = rest[:2 * NBUF_K]
    rows = rest[2 * NBUF_K:3 * NBUF_K]
    deg_v = rest[3 * NBUF_K]
    agg_sh = rest[3 * NBUF_K + 1]
    sems = rest[3 * NBUF_K + 2:]
    bufs = tuple((rows[k], csd[2 * k], csd[2 * k + 1],
                  sems[k], sems[NBUF_K + k]) for k in range(NBUF_K))
    rows0 = rows[0]
    c = lax.axis_index("c")
    s = lax.axis_index("s")
    wid = c * NS_K + s

    pltpu.sync_copy(src_hbm.at[wid], src_b.at[pl.ds(0, N_ROWS_K)])
    pltpu.sync_copy(dst_hbm.at[wid], dst_b.at[pl.ds(0, N_ROWS_K)])
    src_b[N_ROWS_K, pl.ds(0, 16)] = jnp.zeros((16,), jnp.int32)
    dst_b[N_ROWS_K, pl.ds(0, 16)] = jnp.zeros((16,), jnp.int32)

    def zero_deg(i, carry):
        deg_v[pl.ds(i * 16, 16)] = jnp.zeros((16,), jnp.float32)
        return carry
    lax.fori_loop(0, DEG_W_K // 16, zero_deg, 0)

    def zero_rows(t, carry):
        rows0[t // 8, pl.ds((t % 8) * 16, 16)] = jnp.zeros((16,), jnp.float32)
        return carry
    lax.fori_loop(0, CHUNK_K * 8, zero_rows, 0)
    base_row = s * STRIPE_K
    for j in range(STRIPE_K // CHUNK_K):
        pltpu.sync_copy(rows0, agg_sh.at[pl.ds(base_row + j * CHUNK_K, CHUNK_K)])
    rem = STRIPE_K - (STRIPE_K // CHUNK_K) * CHUNK_K
    pltpu.sync_copy(rows0.at[pl.ds(0, rem)],
                    agg_sh.at[pl.ds(base_row + STRIPE_K - rem, rem)])
    plsc.subcore_barrier()

    ones16 = jnp.ones((16,), jnp.float32)
    iota16 = lax.iota(jnp.int32, 16)

    def pre_row(i, cnt):
        for j in range(ROW_K // 16):
            d16 = dst_b[i, pl.ds(j * 16, 16)]
            mask = d16 < TRASH_K
            packed = i * 128 + j * 16 + iota16
            plsc.store_compressed(ids_b.at[pl.ds(cnt, 16)], packed, mask=mask)
            plsc.addupdate_scatter(deg_v, [jnp.minimum(d16, TRASH_K)], ones16)
            cnt = cnt + plsc.all_reduce_population_count(mask)[0]
        return cnt
    cnt = lax.fori_loop(0, N_ROWS_K, pre_row, jnp.int32(0))

    dummy16 = jnp.full((16,), N_ROWS_K * 128, jnp.int32) + iota16
    for k in range(NBUF_K * CHUNK_K // 16 + 1):
        ids_b[pl.ds(cnt + k * 16, 16)] = dummy16
    n_groups = lax.div(cnt + (NBUF_K * CHUNK_K - 1),
                       jnp.int32(NBUF_K * CHUNK_K))
    n_chunks = n_groups * NBUF_K
    pad_cnt = n_chunks * CHUNK_K - cnt
    deg_v[pl.ds(TRASH_K, 16)] = jnp.broadcast_to(
        pad_cnt.astype(jnp.float32), (16,))

    def assemble(i, cs, cd):
        for j in range(CHUNK_K // 16):
            idv = ids_b[pl.ds(i * CHUNK_K + j * 16, 16)]
            rowv = lax.shift_right_logical(idv, 7)
            colv = lax.bitwise_and(idv, 127)
            cs[pl.ds(j * 16, 16)] = plsc.load_gather(src_b, [rowv, colv])
            cd[pl.ds(j * 16, 16)] = plsc.load_gather(dst_b, [rowv, colv])

    def chunk_work(i, p):
        rows_p, cs_p, cd_p, g_p, s_p = bufs[p]
        rows_f, cs_f, cd_f, g_f, s_f = bufs[(p + GDEPTH_K) % NBUF_K]
        pltpu.make_async_copy(x_hbm.at[cs_p], rows_p, g_p).wait()
        pltpu.async_copy(rows_p, agg_sh.at[cd_p], s_p, add=True)

        @pl.when(i >= NBUF_K - GDEPTH_K)
        def _():
            pltpu.make_async_copy(rows_f, agg_sh.at[cd_f], s_f).wait()

        @pl.when(i + GDEPTH_K < n_chunks)
        def _():
            assemble(i + GDEPTH_K, cs_f, cd_f)
            pltpu.async_copy(x_hbm.at[cs_f], rows_f, g_f)

    for d in range(GDEPTH_K):
        @pl.when(d < n_chunks)
        def _(d=d):
            rows_d, cs_d, cd_d, g_d, _ = bufs[d]
            assemble(d, cs_d, cd_d)
            pltpu.async_copy(x_hbm.at[cs_d], rows_d, g_d)

    def body(g, carry):
        for p in range(NBUF_K):
            chunk_work(g * NBUF_K + p, p)
        return carry
    lax.fori_loop(0, n_groups, body, 0)

    for k in range(1, NBUF_K - GDEPTH_K + 1):
        @pl.when(n_chunks >= k)
        def _(k=k):
            rows_q, _, cd_q, _, s_q = bufs[NBUF_K - k]
            pltpu.make_async_copy(rows_q, agg_sh.at[cd_q], s_q).wait()
    plsc.subcore_barrier()

    out_base = s * OUT_STRIPE_K
    pltpu.sync_copy(agg_sh.at[pl.ds(out_base, OUT_STRIPE_K)],
                    agg_out.at[c, pl.ds(out_base, OUT_STRIPE_K)])
    pltpu.sync_copy(deg_v.at[pl.ds(0, DEG_W_K)], deg_out.at[wid])


def _sc_agg(x, edge_index):
    mesh = plsc.VectorSubcoreMesh(core_axis_name="c", subcore_axis_name="s")
    idx3 = edge_index.reshape(2, NW_K, N_ROWS_K, ROW_K)
    return pl.kernel(
        _sc_body,
        out_type=[
            jax.ShapeDtypeStruct((NC_K, N_PAD_K, GDIM), jnp.float32),
            jax.ShapeDtypeStruct((NW_K, DEG_W_K), jnp.float32),
        ],
        mesh=mesh,
        scratch_types=[
            pltpu.VMEM((N_ROWS_K + 1, ROW_K), jnp.int32),
            pltpu.VMEM((N_ROWS_K + 1, ROW_K), jnp.int32),
            pltpu.VMEM((EDGES_PER_TILE_K + NBUF_K * CHUNK_K + 16,), jnp.int32),
        ] + [pltpu.VMEM((CHUNK_K,), jnp.int32)] * (2 * NBUF_K)
        + [pltpu.VMEM((CHUNK_K, GDIM), jnp.float32)] * NBUF_K
        + [
            pltpu.VMEM((DEG_W_K,), jnp.float32),
            pltpu.VMEM_SHARED((SP_ROWS_K, GDIM), jnp.float32),
        ] + [pltpu.SemaphoreType.DMA] * (2 * NBUF_K),
        compiler_params=pltpu.CompilerParams(needs_layout_passes=False),
    )(x, idx3[0], idx3[1])


def _final_body(x_ref, agg_ref, deg_ref, ws_ref, wn_ref, b_ref, o_ref):
    aggs = agg_ref[0] + agg_ref[1]
    degs = deg_ref[...]
    pads = jnp.sum(degs[:, TRASH_K])
    row0 = lax.broadcasted_iota(jnp.int32, (N_PAD_K, 1), 0) == 0
    aggs = aggs - jnp.where(row0, pads, 0.0) * x_ref[0:1, :]
    deg = jnp.maximum(jnp.sum(degs[:, :N_PAD_K], axis=0), 1.0)
    agg = aggs / deg[:, None]
    res = jnp.maximum(
        jnp.dot(x_ref[...], ws_ref[...], preferred_element_type=jnp.float32)
        + jnp.dot(agg, wn_ref[...], preferred_element_type=jnp.float32)
        + b_ref[...], 0.0)
    o_ref[...] = res[:N_OUT_K]


def _final(x, agg2, deg32, w_self, w_neigh, b_graph):
    whole = lambda shape: pl.BlockSpec(shape, lambda i: tuple(0 for _ in shape))
    return pl.pallas_call(
        _final_body,
        grid=(1,),
        in_specs=[
            pl.BlockSpec((N_PAD_K, GDIM), lambda i: (0, 0)),
            whole((NC_K, N_PAD_K, GDIM)),
            whole((NW_K, DEG_W_K)),
            whole((GDIM, GDIM)),
            whole((GDIM, GDIM)),
            whole((1, GDIM)),
        ],
        out_specs=pl.BlockSpec((N_OUT_K, GDIM), lambda i: (0, 0)),
        out_shape=jax.ShapeDtypeStruct((N_OUT_K, GDIM), jnp.float32),
    )(x, agg2, deg32, w_self, w_neigh, b_graph.reshape(1, GDIM))


def kernel(x_customer, x_product, edge_index, lin_c_w, lin_c_b,
           lin_p_w, lin_p_b, w_self, w_neigh, b_graph):
    x = _proj(x_customer, x_product, lin_c_w, lin_c_b, lin_p_w, lin_p_b)
    agg2, deg32 = _sc_agg(x, edge_index)
    return _final(x, agg2, deg32, w_self, w_neigh, b_graph)


# speedup vs baseline: 1.4160x; 1.4160x over previous
import jax
import jax.numpy as jnp
from jax import lax
from jax.experimental import pallas as pl
from jax.experimental.pallas import tpu as pltpu
from jax.experimental.pallas import tpu_sc as plsc

N_NODES_K = 10000
N_OUT_K = 6000
GDIM = 128
E_K = 320000
NC_K, NS_K = 2, 16
NW_K = NC_K * NS_K
CHUNK_K = 32
CHUNK_SHIFT_K = 5
NBUF_K = 7
GDEPTH_K = 6
EDGES_PER_TILE_K = E_K // NW_K
ROW_K = 80
N_ROWS_K = EDGES_PER_TILE_K // ROW_K
SP_ROWS_K = 6016
STRIPE_K = SP_ROWS_K // NS_K
TRASH_K = 6016
DEG_W_K = 6032
N_PAD_K = 6016
OUT_STRIPE_K = N_PAD_K // NS_K


_PROJ_BLOCK = 1000


def _proj_body(xc_ref, xp_ref, wc_ref, bc_ref, wp_ref, bp_ref, o_ref):
    i = pl.program_id(0)

    @pl.when(i < 6)
    def _():
        o_ref[...] = jnp.dot(xc_ref[...], wc_ref[...],
                             preferred_element_type=jnp.float32) + bc_ref[...]

    @pl.when(i >= 6)
    def _():
        o_ref[...] = jnp.dot(xp_ref[...], wp_ref[...],
                             preferred_element_type=jnp.float32) + bp_ref[...]


def _proj(x_customer, x_product, lin_c_w, lin_c_b, lin_p_w, lin_p_b):
    full = pl.BlockSpec((GDIM, GDIM), lambda i: (0, 0))
    bias = pl.BlockSpec((1, GDIM), lambda i: (0, 0))
    return pl.pallas_call(
        _proj_body,
        grid=(N_NODES_K // _PROJ_BLOCK,),
        in_specs=[
            pl.BlockSpec((_PROJ_BLOCK, GDIM), lambda i: (jnp.minimum(i, 5), 0)),
            pl.BlockSpec((_PROJ_BLOCK, GDIM), lambda i: (jnp.maximum(i - 6, 0), 0)),
            full, bias, full, bias,
        ],
        out_specs=pl.BlockSpec((_PROJ_BLOCK, GDIM), lambda i: (i, 0)),
        out_shape=jax.ShapeDtypeStruct((N_NODES_K, GDIM), jnp.float32),
    )(x_customer, x_product, lin_c_w, lin_c_b.reshape(1, GDIM),
      lin_p_w, lin_p_b.reshape(1, GDIM))


def _sc_body(x_hbm, src_hbm, dst_hbm, agg_out, deg_out,
             src_b, dst_b, ids_b, *rest):
    csd = rest[:2 * NBUF_K]
    rows = rest[2 * NBUF_K:3 * NBUF_K]
    deg_v = rest[3 * NBUF_K]
    agg_sh = rest[3 * NBUF_K + 1]
    sems = rest[3 * NBUF_K + 2:]
    bufs = tuple((rows[k], csd[2 * k], csd[2 * k + 1],
                  sems[k], sems[NBUF_K + k]) for k in range(NBUF_K))
    rows0 = rows[0]
    c = lax.axis_index("c")
    s = lax.axis_index("s")
    wid = c * NS_K + s

    pltpu.sync_copy(src_hbm.at[wid], src_b.at[pl.ds(0, N_ROWS_K)])
    pltpu.sync_copy(dst_hbm.at[wid], dst_b.at[pl.ds(0, N_ROWS_K)])
    src_b[N_ROWS_K, pl.ds(0, 16)] = jnp.zeros((16,), jnp.int32)
    dst_b[N_ROWS_K, pl.ds(0, 16)] = jnp.zeros((16,), jnp.int32)

    def zero_deg(i, carry):
        deg_v[pl.ds(i * 16, 16)] = jnp.zeros((16,), jnp.float32)
        return carry
    lax.fori_loop(0, DEG_W_K // 16, zero_deg, 0)

    def zero_rows(t, carry):
        rows0[t // 8, pl.ds((t % 8) * 16, 16)] = jnp.zeros((16,), jnp.float32)
        return carry
    lax.fori_loop(0, CHUNK_K * 8, zero_rows, 0)
    base_row = s * STRIPE_K
    for j in range(STRIPE_K // CHUNK_K):
        pltpu.sync_copy(rows0, agg_sh.at[pl.ds(base_row + j * CHUNK_K, CHUNK_K)])
    rem = STRIPE_K - (STRIPE_K // CHUNK_K) * CHUNK_K
    pltpu.sync_copy(rows0.at[pl.ds(0, rem)],
                    agg_sh.at[pl.ds(base_row + STRIPE_K - rem, rem)])
    plsc.subcore_barrier()

    ones16 = jnp.ones((16,), jnp.float32)
    iota16 = lax.iota(jnp.int32, 16)

    def pre_row(i, cnt):
        for j in range(ROW_K // 16):
            d16 = dst_b[i, pl.ds(j * 16, 16)]
            mask = d16 < TRASH_K
            packed = i * 128 + j * 16 + iota16
            plsc.store_compressed(ids_b.at[pl.ds(cnt, 16)], packed, mask=mask)
            plsc.addupdate_scatter(deg_v, [jnp.minimum(d16, TRASH_K)], ones16)
            cnt = cnt + plsc.all_reduce_population_count(mask)[0]
        return cnt
    cnt = lax.fori_loop(0, N_ROWS_K, pre_row, jnp.int32(0))

    dummy16 = jnp.full((16,), N_ROWS_K * 128, jnp.int32) + iota16
    for k in range(CHUNK_K // 16):
        ids_b[pl.ds(cnt + k * 16, 16)] = dummy16
    n_chunks = lax.shift_right_logical(cnt + (CHUNK_K - 1), CHUNK_SHIFT_K)
    pad_cnt = n_chunks * CHUNK_K - cnt
    deg_v[pl.ds(TRASH_K, 16)] = jnp.broadcast_to(
        pad_cnt.astype(jnp.float32), (16,))

    def assemble(i, cs, cd):
        for j in range(CHUNK_K // 16):
            idv = ids_b[pl.ds(i * CHUNK_K + j * 16, 16)]
            rowv = lax.shift_right_logical(idv, 7)
            colv = lax.bitwise_and(idv, 127)
            cs[pl.ds(j * 16, 16)] = plsc.load_gather(src_b, [rowv, colv])
            cd[pl.ds(j * 16, 16)] = plsc.load_gather(dst_b, [rowv, colv])

    def chunk_work(i, p):
        rows_p, cs_p, cd_p, g_p, s_p = bufs[p]
        rows_f, cs_f, cd_f, g_f, s_f = bufs[(p + GDEPTH_K) % NBUF_K]
        pltpu.make_async_copy(x_hbm.at[cs_p], rows_p, g_p).wait()
        pltpu.async_copy(rows_p, agg_sh.at[cd_p], s_p, add=True)

        @pl.when(i >= NBUF_K - GDEPTH_K)
        def _():
            pltpu.make_async_copy(rows_f, agg_sh.at[cd_f], s_f).wait()

        @pl.when(i + GDEPTH_K < n_chunks)
        def _():
            assemble(i + GDEPTH_K, cs_f, cd_f)
            pltpu.async_copy(x_hbm.at[cs_f], rows_f, g_f)

    for d in range(GDEPTH_K):
        @pl.when(d < n_chunks)
        def _(d=d):
            rows_d, cs_d, cd_d, g_d, _ = bufs[d]
            assemble(d, cs_d, cd_d)
            pltpu.async_copy(x_hbm.at[cs_d], rows_d, g_d)

    def body(i, carry):
        for p in range(NBUF_K):
            @pl.when(lax.rem(i, NBUF_K) == p)
            def _(p=p):
                chunk_work(i, p)
        return carry
    lax.fori_loop(0, n_chunks, body, 0)

    for k in range(1, NBUF_K - GDEPTH_K + 1):
        jk = n_chunks - k
        for q in range(NBUF_K):
            @pl.when(jnp.logical_and(jk >= 0, lax.rem(jk, NBUF_K) == q))
            def _(q=q):
                rows_q, _, cd_q, _, s_q = bufs[q]
                pltpu.make_async_copy(rows_q, agg_sh.at[cd_q], s_q).wait()
    plsc.subcore_barrier()

    out_base = s * OUT_STRIPE_K
    pltpu.sync_copy(agg_sh.at[pl.ds(out_base, OUT_STRIPE_K)],
                    agg_out.at[c, pl.ds(out_base, OUT_STRIPE_K)])
    pltpu.sync_copy(deg_v.at[pl.ds(0, DEG_W_K)], deg_out.at[wid])


def _sc_agg(x, edge_index):
    mesh = plsc.VectorSubcoreMesh(core_axis_name="c", subcore_axis_name="s")
    idx3 = edge_index.reshape(2, NW_K, N_ROWS_K, ROW_K)
    return pl.kernel(
        _sc_body,
        out_type=[
            jax.ShapeDtypeStruct((NC_K, N_PAD_K, GDIM), jnp.float32),
            jax.ShapeDtypeStruct((NW_K, DEG_W_K), jnp.float32),
        ],
        mesh=mesh,
        scratch_types=[
            pltpu.VMEM((N_ROWS_K + 1, ROW_K), jnp.int32),
            pltpu.VMEM((N_ROWS_K + 1, ROW_K), jnp.int32),
            pltpu.VMEM((EDGES_PER_TILE_K + NBUF_K * CHUNK_K + 16,), jnp.int32),
        ] + [pltpu.VMEM((CHUNK_K,), jnp.int32)] * (2 * NBUF_K)
        + [pltpu.VMEM((CHUNK_K, GDIM), jnp.float32)] * NBUF_K
        + [
            pltpu.VMEM((DEG_W_K,), jnp.float32),
            pltpu.VMEM_SHARED((SP_ROWS_K, GDIM), jnp.float32),
        ] + [pltpu.SemaphoreType.DMA] * (2 * NBUF_K),
        compiler_params=pltpu.CompilerParams(needs_layout_passes=False),
    )(x, idx3[0], idx3[1])


def _final_body(x_ref, agg_ref, deg_ref, ws_ref, wn_ref, b_ref, o_ref):
    aggs = agg_ref[0] + agg_ref[1]
    degs = deg_ref[...]
    pads = jnp.sum(degs[:, TRASH_K])
    row0 = lax.broadcasted_iota(jnp.int32, (N_PAD_K, 1), 0) == 0
    aggs = aggs - jnp.where(row0, pads, 0.0) * x_ref[0:1, :]
    deg = jnp.maximum(jnp.sum(degs[:, :N_PAD_K], axis=0), 1.0)
    agg = aggs / deg[:, None]
    res = jnp.maximum(
        jnp.dot(x_ref[...], ws_ref[...], preferred_element_type=jnp.float32)
        + jnp.dot(agg, wn_ref[...], preferred_element_type=jnp.float32)
        + b_ref[...], 0.0)
    o_ref[...] = res[:N_OUT_K]


def _final(x, agg2, deg32, w_self, w_neigh, b_graph):
    whole = lambda shape: pl.BlockSpec(shape, lambda i: tuple(0 for _ in shape))
    return pl.pallas_call(
        _final_body,
        grid=(1,),
        in_specs=[
            pl.BlockSpec((N_PAD_K, GDIM), lambda i: (0, 0)),
            whole((NC_K, N_PAD_K, GDIM)),
            whole((NW_K, DEG_W_K)),
            whole((GDIM, GDIM)),
            whole((GDIM, GDIM)),
            whole((1, GDIM)),
        ],
        out_specs=pl.BlockSpec((N_OUT_K, GDIM), lambda i: (0, 0)),
        out_shape=jax.ShapeDtypeStruct((N_OUT_K, GDIM), jnp.float32),
    )(x, agg2, deg32, w_self, w_neigh, b_graph.reshape(1, GDIM))


def kernel(x_customer, x_product, edge_index, lin_c_w, lin_c_b,
           lin_p_w, lin_p_b, w_self, w_neigh, b_graph):
    x = _proj(x_customer, x_product, lin_c_w, lin_c_b, lin_p_w, lin_p_b)
    agg2, deg32 = _sc_agg(x, edge_index)
    return _final(x, agg2, deg32, w_self, w_neigh, b_graph)


# speedup vs baseline: 1.4257x; 1.0068x over previous
import jax
import jax.numpy as jnp
from jax import lax
from jax.experimental import pallas as pl
from jax.experimental.pallas import tpu as pltpu
from jax.experimental.pallas import tpu_sc as plsc

N_NODES_K = 10000
N_OUT_K = 6000
GDIM = 128
E_K = 320000
NC_K, NS_K = 2, 16
NW_K = NC_K * NS_K
CHUNK_K = 32
CHUNK_SHIFT_K = 5
NBUF_K = 7
GDEPTH_K = 6
EDGES_PER_TILE_K = E_K // NW_K
ROW_K = 80
N_ROWS_K = EDGES_PER_TILE_K // ROW_K
SP_ROWS_K = 6016
STRIPE_K = SP_ROWS_K // NS_K
TRASH_K = 6016
DEG_W_K = 6032
N_PAD_K = 6016
OUT_STRIPE_K = N_PAD_K // NS_K


_PROJ_BLOCK = 1000


def _proj_body(xc_ref, xp_ref, wc_ref, bc_ref, wp_ref, bp_ref, o_ref):
    i = pl.program_id(0)

    @pl.when(i < 6)
    def _():
        o_ref[...] = jnp.dot(xc_ref[...], wc_ref[...],
                             preferred_element_type=jnp.float32) + bc_ref[...]

    @pl.when(i >= 6)
    def _():
        o_ref[...] = jnp.dot(xp_ref[...], wp_ref[...],
                             preferred_element_type=jnp.float32) + bp_ref[...]


def _proj(x_customer, x_product, lin_c_w, lin_c_b, lin_p_w, lin_p_b):
    full = pl.BlockSpec((GDIM, GDIM), lambda i: (0, 0))
    bias = pl.BlockSpec((1, GDIM), lambda i: (0, 0))
    return pl.pallas_call(
        _proj_body,
        grid=(N_NODES_K // _PROJ_BLOCK,),
        in_specs=[
            pl.BlockSpec((_PROJ_BLOCK, GDIM), lambda i: (jnp.minimum(i, 5), 0)),
            pl.BlockSpec((_PROJ_BLOCK, GDIM), lambda i: (jnp.maximum(i - 6, 0), 0)),
            full, bias, full, bias,
        ],
        out_specs=pl.BlockSpec((_PROJ_BLOCK, GDIM), lambda i: (i, 0)),
        out_shape=jax.ShapeDtypeStruct((N_NODES_K, GDIM), jnp.float32),
    )(x_customer, x_product, lin_c_w, lin_c_b.reshape(1, GDIM),
      lin_p_w, lin_p_b.reshape(1, GDIM))


def _sc_body(x_hbm, src_hbm, dst_hbm, agg_out, deg_out,
             src_b, dst_b, ids_b, *rest):
    csd = rest[:2 * NBUF_K]
    rows = rest[2 * NBUF_K:3 * NBUF_K]
    deg_v = rest[3 * NBUF_K]
    agg_sh = rest[3 * NBUF_K + 1]
    sems = rest[3 * NBUF_K + 2:]
    bufs = tuple((rows[k], csd[2 * k], csd[2 * k + 1],
                  sems[k], sems[NBUF_K + k]) for k in range(NBUF_K))
    rows0 = rows[0]
    c = lax.axis_index("c")
    s = lax.axis_index("s")
    wid = c * NS_K + s

    pltpu.sync_copy(src_hbm.at[wid], src_b.at[pl.ds(0, N_ROWS_K)])
    pltpu.sync_copy(dst_hbm.at[wid], dst_b.at[pl.ds(0, N_ROWS_K)])
    src_b[N_ROWS_K, pl.ds(0, 16)] = jnp.zeros((16,), jnp.int32)
    dst_b[N_ROWS_K, pl.ds(0, 16)] = jnp.zeros((16,), jnp.int32)

    def zero_deg(i, carry):
        deg_v[pl.ds(i * 16, 16)] = jnp.zeros((16,), jnp.float32)
        return carry
    lax.fori_loop(0, DEG_W_K // 16, zero_deg, 0)

    def zero_rows(t, carry):
        rows0[t // 8, pl.ds((t % 8) * 16, 16)] = jnp.zeros((16,), jnp.float32)
        return carry
    lax.fori_loop(0, CHUNK_K * 8, zero_rows, 0)
    base_row = s * STRIPE_K
    rem = STRIPE_K - (STRIPE_K // CHUNK_K) * CHUNK_K
    for j in range(STRIPE_K // CHUNK_K):
        pltpu.async_copy(rows0, agg_sh.at[pl.ds(base_row + j * CHUNK_K, CHUNK_K)],
                         sems[0])
    pltpu.async_copy(rows0.at[pl.ds(0, rem)],
                     agg_sh.at[pl.ds(base_row + STRIPE_K - rem, rem)], sems[0])
    for j in range(STRIPE_K // CHUNK_K):
        pltpu.make_async_copy(rows0, agg_sh.at[pl.ds(base_row + j * CHUNK_K, CHUNK_K)],
                              sems[0]).wait()
    pltpu.make_async_copy(rows0.at[pl.ds(0, rem)],
                          agg_sh.at[pl.ds(base_row + STRIPE_K - rem, rem)],
                          sems[0]).wait()
    plsc.subcore_barrier()

    ones16 = jnp.ones((16,), jnp.float32)
    iota16 = lax.iota(jnp.int32, 16)

    def pre_row(i, cnt):
        for j in range(ROW_K // 16):
            d16 = dst_b[i, pl.ds(j * 16, 16)]
            mask = d16 < TRASH_K
            packed = i * 128 + j * 16 + iota16
            plsc.store_compressed(ids_b.at[pl.ds(cnt, 16)], packed, mask=mask)
            plsc.addupdate_scatter(deg_v, [jnp.minimum(d16, TRASH_K)], ones16)
            cnt = cnt + plsc.all_reduce_population_count(mask)[0]
        return cnt
    cnt = lax.fori_loop(0, N_ROWS_K, pre_row, jnp.int32(0))

    dummy16 = jnp.full((16,), N_ROWS_K * 128, jnp.int32) + iota16
    for k in range(CHUNK_K // 16):
        ids_b[pl.ds(cnt + k * 16, 16)] = dummy16
    n_chunks = lax.shift_right_logical(cnt + (CHUNK_K - 1), CHUNK_SHIFT_K)
    pad_cnt = n_chunks * CHUNK_K - cnt
    deg_v[pl.ds(TRASH_K, 16)] = jnp.broadcast_to(
        pad_cnt.astype(jnp.float32), (16,))

    def assemble(i, cs, cd):
        for j in range(CHUNK_K // 16):
            idv = ids_b[pl.ds(i * CHUNK_K + j * 16, 16)]
            rowv = lax.shift_right_logical(idv, 7)
            colv = lax.bitwise_and(idv, 127)
            cs[pl.ds(j * 16, 16)] = plsc.load_gather(src_b, [rowv, colv])
            cd[pl.ds(j * 16, 16)] = plsc.load_gather(dst_b, [rowv, colv])

    def chunk_work(i, p):
        rows_p, cs_p, cd_p, g_p, s_p = bufs[p]
        rows_f, cs_f, cd_f, g_f, s_f = bufs[(p + GDEPTH_K) % NBUF_K]
        pltpu.make_async_copy(x_hbm.at[cs_p], rows_p, g_p).wait()
        pltpu.async_copy(rows_p, agg_sh.at[cd_p], s_p, add=True)

        @pl.when(i >= NBUF_K - GDEPTH_K)
        def _():
            pltpu.make_async_copy(rows_f, agg_sh.at[cd_f], s_f).wait()

        @pl.when(i + GDEPTH_K < n_chunks)
        def _():
            assemble(i + GDEPTH_K, cs_f, cd_f)
            pltpu.async_copy(x_hbm.at[cs_f], rows_f, g_f)

    for d in range(GDEPTH_K):
        @pl.when(d < n_chunks)
        def _(d=d):
            rows_d, cs_d, cd_d, g_d, _ = bufs[d]
            assemble(d, cs_d, cd_d)
            pltpu.async_copy(x_hbm.at[cs_d], rows_d, g_d)

    def body(i, carry):
        for p in range(NBUF_K):
            @pl.when(lax.rem(i, NBUF_K) == p)
            def _(p=p):
                chunk_work(i, p)
        return carry
    lax.fori_loop(0, n_chunks, body, 0)

    for k in range(1, NBUF_K - GDEPTH_K + 1):
        jk = n_chunks - k
        for q in range(NBUF_K):
            @pl.when(jnp.logical_and(jk >= 0, lax.rem(jk, NBUF_K) == q))
            def _(q=q):
                rows_q, _, cd_q, _, s_q = bufs[q]
                pltpu.make_async_copy(rows_q, agg_sh.at[cd_q], s_q).wait()
    plsc.subcore_barrier()

    out_base = s * OUT_STRIPE_K
    pltpu.async_copy(agg_sh.at[pl.ds(out_base, OUT_STRIPE_K)],
                     agg_out.at[c, pl.ds(out_base, OUT_STRIPE_K)], sems[0])
    pltpu.async_copy(deg_v.at[pl.ds(0, DEG_W_K)], deg_out.at[wid], sems[1])
    pltpu.make_async_copy(agg_sh.at[pl.ds(out_base, OUT_STRIPE_K)],
                          agg_out.at[c, pl.ds(out_base, OUT_STRIPE_K)],
                          sems[0]).wait()
    pltpu.make_async_copy(deg_v.at[pl.ds(0, DEG_W_K)], deg_out.at[wid],
                          sems[1]).wait()


def _sc_agg(x, edge_index):
    mesh = plsc.VectorSubcoreMesh(core_axis_name="c", subcore_axis_name="s")
    idx3 = edge_index.reshape(2, NW_K, N_ROWS_K, ROW_K)
    return pl.kernel(
        _sc_body,
        out_type=[
            jax.ShapeDtypeStruct((NC_K, N_PAD_K, GDIM), jnp.float32),
            jax.ShapeDtypeStruct((NW_K, DEG_W_K), jnp.float32),
        ],
        mesh=mesh,
        scratch_types=[
            pltpu.VMEM((N_ROWS_K + 1, ROW_K), jnp.int32),
            pltpu.VMEM((N_ROWS_K + 1, ROW_K), jnp.int32),
            pltpu.VMEM((EDGES_PER_TILE_K + NBUF_K * CHUNK_K + 16,), jnp.int32),
        ] + [pltpu.VMEM((CHUNK_K,), jnp.int32)] * (2 * NBUF_K)
        + [pltpu.VMEM((CHUNK_K, GDIM), jnp.float32)] * NBUF_K
        + [
            pltpu.VMEM((DEG_W_K,), jnp.float32),
            pltpu.VMEM_SHARED((SP_ROWS_K, GDIM), jnp.float32),
        ] + [pltpu.SemaphoreType.DMA] * (2 * NBUF_K),
        compiler_params=pltpu.CompilerParams(needs_layout_passes=False),
    )(x, idx3[0], idx3[1])


def _final_body(x_ref, agg_ref, deg_ref, ws_ref, wn_ref, b_ref, o_ref):
    aggs = agg_ref[0] + agg_ref[1]
    degs = deg_ref[...]
    pads = jnp.sum(degs[:, TRASH_K])
    row0 = lax.broadcasted_iota(jnp.int32, (N_PAD_K, 1), 0) == 0
    aggs = aggs - jnp.where(row0, pads, 0.0) * x_ref[0:1, :]
    deg = jnp.maximum(jnp.sum(degs[:, :N_PAD_K], axis=0), 1.0)
    agg = aggs / deg[:, None]
    res = jnp.maximum(
        jnp.dot(x_ref[...], ws_ref[...], preferred_element_type=jnp.float32)
        + jnp.dot(agg, wn_ref[...], preferred_element_type=jnp.float32)
        + b_ref[...], 0.0)
    o_ref[...] = res[:N_OUT_K]


def _final(x, agg2, deg32, w_self, w_neigh, b_graph):
    whole = lambda shape: pl.BlockSpec(shape, lambda i: tuple(0 for _ in shape))
    return pl.pallas_call(
        _final_body,
        grid=(1,),
        in_specs=[
            pl.BlockSpec((N_PAD_K, GDIM), lambda i: (0, 0)),
            whole((NC_K, N_PAD_K, GDIM)),
            whole((NW_K, DEG_W_K)),
            whole((GDIM, GDIM)),
            whole((GDIM, GDIM)),
            whole((1, GDIM)),
        ],
        out_specs=pl.BlockSpec((N_OUT_K, GDIM), lambda i: (0, 0)),
        out_shape=jax.ShapeDtypeStruct((N_OUT_K, GDIM), jnp.float32),
    )(x, agg2, deg32, w_self, w_neigh, b_graph.reshape(1, GDIM))


def kernel(x_customer, x_product, edge_index, lin_c_w, lin_c_b,
           lin_p_w, lin_p_b, w_self, w_neigh, b_graph):
    x = _proj(x_customer, x_product, lin_c_w, lin_c_b, lin_p_w, lin_p_b)
    agg2, deg32 = _sc_agg(x, edge_index)
    return _final(x, agg2, deg32, w_self, w_neigh, b_graph)
